# baseline (speedup 1.0000x reference)
import jax
import jax.numpy as jnp
from jax import lax
from jax.experimental import pallas as pl
from jax.experimental.pallas import tpu as pltpu
from jax.experimental.pallas import tpu_sc as plsc

N = 2048
I = 256
O = 256
B = 512
BS = 128
L = (N - I) // BS
NH = N - I
E = NH * 32

R0 = [I + BS * l for l in range(L)]
OFF128 = [64 * l * l + 192 * l for l in range(L)]
OFF = [o * BS for o in OFF128]
TOT = OFF[-1] + R0[-1] * BS

NC = 2
NS = 16
CUT = 983040
SIZE0 = CUT
SIZE1 = TOT - CUT
PAD = 64
OUT_FLOATS = TOT
EPW = E // NS
ROWS128 = EPW // 128
TPC0 = SIZE0 // NS
TPC1 = SIZE1 // NS
ZB = 12288
ZITER = TPC0 // ZB


def _sc_scatter_body(widx_hbm, val_hbm, out_hbm,
                     dstw_v, srcw_v, val_v, idx_v, zbuf_v, wsp,
                     ldsem, zsem, ssem):
  c = lax.axis_index("c")
  s = lax.axis_index("s")

  ld_d = pltpu.async_copy(widx_hbm.at[0, pl.ds(s * EPW, EPW)], dstw_v, ldsem)
  ld_s = pltpu.async_copy(widx_hbm.at[1, pl.ds(s * EPW, EPW)], srcw_v, ldsem)
  ld_v = pltpu.async_copy(val_hbm.at[pl.ds(s * EPW, EPW)], val_v, ldsem)

  @plsc.parallel_loop(0, ZB, step=16, unroll=4)
  def _zfill(i):
    zbuf_v[pl.ds(i, 16)] = jnp.zeros((16,), jnp.float32)

  zcps = []
  for j in range(ZITER):
    zcps.append(pltpu.async_copy(
        zbuf_v, wsp.at[pl.ds(s * TPC0 + j * ZB, ZB)], zsem))

  ld_d.wait()
  ld_s.wait()
  base = c * CUT
  size_c = jnp.where(c == 0, SIZE0, SIZE1)
  trash = TPC0 * NS + s * 4

  @plsc.parallel_loop(0, ROWS128 * 8, step=1, unroll=4)
  def _cvt(i):
    j = i // 8
    k = i - j * 8
    d = dstw_v[pl.ds(i * 16, 16)] - I
    sr = srcw_v[pl.ds(i * 16, 16)]
    lyr = jnp.right_shift(d, 7)
    flat = (8192 * lyr + 24576) * lyr + sr * 128 + (d - 128 * lyr) - base
    ok = (flat >= 0) & (flat < size_c)
    idx_v[j, pl.ds(k * 16, 16)] = jnp.where(ok, flat, trash)

  ld_v.wait()
  for cp in zcps:
    cp.wait()
  plsc.subcore_barrier()

  cps = [pltpu.async_copy(val_v.at[pl.ds(j * 128, 128)],
                          wsp.at[idx_v.at[j]], ssem, add=True)
         for j in range(ROWS128)]
  for cp in cps:
    cp.wait()
  plsc.subcore_barrier()

  @pl.when(c == 0)
  def _out0():
    pltpu.sync_copy(wsp.at[pl.ds(s * TPC0, TPC0)],
                    out_hbm.at[pl.ds(s * TPC0, TPC0)])

  @pl.when(c == 1)
  def _out1():
    pltpu.sync_copy(wsp.at[pl.ds(s * TPC1, TPC1)],
                    out_hbm.at[pl.ds(CUT + s * TPC1, TPC1)])


def _sc_scatter(W_idx, W_val):
  mesh = plsc.VectorSubcoreMesh(core_axis_name="c", subcore_axis_name="s")
  return pl.kernel(
      _sc_scatter_body,
      out_type=jax.ShapeDtypeStruct((OUT_FLOATS,), jnp.float32),
      mesh=mesh,
      scratch_types=[
          pltpu.VMEM((EPW,), jnp.int32),
          pltpu.VMEM((EPW,), jnp.int32),
          pltpu.VMEM((EPW,), jnp.float32),
          pltpu.VMEM((ROWS128, 128), jnp.int32),
          pltpu.VMEM((ZB,), jnp.float32),
          pltpu.VMEM_SHARED((CUT + PAD,), jnp.float32),
          pltpu.SemaphoreType.DMA,
          pltpu.SemaphoreType.DMA,
          pltpu.SemaphoreType.DMA,
      ],
  )(W_idx, W_val)


def _tc_scan_body(x_ref, w_hbm, b_ref, o_ref, a_scr, *rest):
  bufs, sems = rest[:L], rest[L:]
  copies = []
  for l in range(L):
    cp = pltpu.make_async_copy(w_hbm.at[pl.ds(OFF128[l], R0[l])],
                               bufs[l], sems[l])
    cp.start()
    copies.append(cp)
  a_scr[:, :I] = x_ref[...].astype(jnp.bfloat16)
  dn = (((1,), (0,)), ((), ()))
  hprev = None
  for l in range(L):
    r0 = R0[l]
    copies[l].wait()
    w16 = bufs[l][...].astype(jnp.bfloat16)
    if l == 0:
      z = lax.dot_general(a_scr[:, :r0], w16, dn,
                          preferred_element_type=jnp.float32)
    else:
      z = (lax.dot_general(a_scr[:, :r0 - BS], w16[:r0 - BS], dn,
                           preferred_element_type=jnp.float32)
           + lax.dot_general(hprev, w16[r0 - BS:], dn,
                             preferred_element_type=jnp.float32))
    h = 1.0 / (1.0 + jnp.exp(-(z + b_ref[pl.ds(I + l * BS, BS)])))
    hprev = h.astype(jnp.bfloat16)
    a_scr[:, r0:r0 + BS] = hprev
    if l >= L - O // BS:
      lo = (l - (L - O // BS)) * BS
      o_ref[:, lo:lo + BS] = h


def _tc_scan(x, w2, b):
  return pl.pallas_call(
      _tc_scan_body,
      in_specs=[
          pl.BlockSpec((B, I), lambda: (0, 0)),
          pl.BlockSpec(memory_space=pltpu.MemorySpace.HBM),
          pl.BlockSpec((N,), lambda: (0,)),
      ],
      out_specs=pl.BlockSpec((B, O), lambda: (0, 0)),
      out_shape=jax.ShapeDtypeStruct((B, O), jnp.float32),
      scratch_shapes=(
          [pltpu.VMEM((B, N), jnp.bfloat16)]
          + [pltpu.VMEM((R0[l], 128), jnp.float32) for l in range(L)]
          + [pltpu.SemaphoreType.DMA for _ in range(L)]
      ),
  )(x, w2, b)


def kernel(x, W_idx, W_val, b):
  wflat = _sc_scatter(W_idx, W_val)
  w2 = wflat.reshape(OUT_FLOATS // 128, 128)
  return _tc_scan(x, w2, b)


# speedup vs baseline: 1.0087x; 1.0087x over previous
import jax
import jax.numpy as jnp
from jax import lax
from jax.experimental import pallas as pl
from jax.experimental.pallas import tpu as pltpu
from jax.experimental.pallas import tpu_sc as plsc

N = 2048
I = 256
O = 256
B = 512
BS = 128
L = (N - I) // BS
NH = N - I
E = NH * 32

R0 = [I + BS * l for l in range(L)]
OFF128 = [64 * l * l + 192 * l for l in range(L)]
OFF = [o * BS for o in OFF128]
TOT = OFF[-1] + R0[-1] * BS

NC = 2
NS = 16
CUT = 983040
SIZE0 = CUT
SIZE1 = TOT - CUT
PAD = 64
OUT_FLOATS = TOT
EPW = E // NS
ROWS128 = EPW // 128
TPC0 = SIZE0 // NS
TPC1 = SIZE1 // NS
ZB = 5120
ZITER = TPC0 // ZB


def _sc_scatter_body(widx_hbm, val_hbm, out_hbm,
                     dstw_v, srcw_v, val_v, idx_v, zbuf_v, wsp,
                     ldsem, zsem, ssem):
  c = lax.axis_index("c")
  s = lax.axis_index("s")

  ld_d = pltpu.async_copy(widx_hbm.at[0, pl.ds(s * EPW, EPW)], dstw_v, ldsem)
  ld_s = pltpu.async_copy(widx_hbm.at[1, pl.ds(s * EPW, EPW)], srcw_v, ldsem)
  ld_v = pltpu.async_copy(val_hbm.at[pl.ds(s * EPW, EPW)], val_v, ldsem)

  @plsc.parallel_loop(0, ZB, step=16, unroll=4)
  def _zfill(i):
    zbuf_v[pl.ds(i, 16)] = jnp.zeros((16,), jnp.float32)

  zcps = []
  for j in range(ZITER):
    zcps.append(pltpu.async_copy(
        zbuf_v, wsp.at[pl.ds(s * TPC0 + j * ZB, ZB)], zsem))

  ld_d.wait()
  ld_s.wait()
  base = c * CUT
  size_c = jnp.where(c == 0, SIZE0, SIZE1)
  trash = TPC0 * NS + s * 4

  @plsc.parallel_loop(0, ROWS128 * 8, step=1, unroll=2)
  def _cvt(i):
    j = i // 8
    k = i - j * 8
    d = dstw_v[pl.ds(i * 16, 16)] - I
    sr = srcw_v[pl.ds(i * 16, 16)]
    lyr = jnp.right_shift(d, 7)
    flat = (8192 * lyr + 24576) * lyr + sr * 128 + (d - 128 * lyr) - base
    ok = (flat >= 0) & (flat < size_c) & (sr < 128 * lyr + I)
    idx_v[j, pl.ds(k * 16, 16)] = jnp.where(ok, flat, trash)

  ld_v.wait()
  for cp in zcps:
    cp.wait()
  plsc.subcore_barrier()

  for j0 in range(0, ROWS128, 14):
    cps = [pltpu.async_copy(val_v.at[pl.ds(j * 128, 128)],
                            wsp.at[idx_v.at[j]], ssem, add=True)
           for j in range(j0, j0 + 14)]
    for cp in cps:
      cp.wait()
  plsc.subcore_barrier()

  @pl.when(c == 0)
  def _out0():
    pltpu.sync_copy(wsp.at[pl.ds(s * TPC0, TPC0)],
                    out_hbm.at[pl.ds(s * TPC0, TPC0)])

  @pl.when(c == 1)
  def _out1():
    pltpu.sync_copy(wsp.at[pl.ds(s * TPC1, TPC1)],
                    out_hbm.at[pl.ds(CUT + s * TPC1, TPC1)])


def _sc_scatter(W_idx, W_val):
  mesh = plsc.VectorSubcoreMesh(core_axis_name="c", subcore_axis_name="s")
  return pl.kernel(
      _sc_scatter_body,
      out_type=jax.ShapeDtypeStruct((OUT_FLOATS,), jnp.float32),
      mesh=mesh,
      scratch_types=[
          pltpu.VMEM((EPW,), jnp.int32),
          pltpu.VMEM((EPW,), jnp.int32),
          pltpu.VMEM((EPW,), jnp.float32),
          pltpu.VMEM((ROWS128, 128), jnp.int32),
          pltpu.VMEM((ZB,), jnp.float32),
          pltpu.VMEM_SHARED((CUT + PAD,), jnp.float32),
          pltpu.SemaphoreType.DMA,
          pltpu.SemaphoreType.DMA,
          pltpu.SemaphoreType.DMA,
      ],
  )(W_idx, W_val)


def _tc_scan_body(x_ref, w_hbm, b_ref, o_ref, a_scr, *rest):
  bufs, sems = rest[:L], rest[L:]
  copies = []
  for l in range(L):
    cp = pltpu.make_async_copy(w_hbm.at[pl.ds(OFF128[l], R0[l])],
                               bufs[l], sems[l])
    cp.start()
    copies.append(cp)
  a_scr[:, :I] = x_ref[...].astype(jnp.bfloat16)
  dn = (((1,), (0,)), ((), ()))
  hprev = None
  for l in range(L):
    r0 = R0[l]
    copies[l].wait()
    w16 = bufs[l][...].astype(jnp.bfloat16)
    if l == 0:
      z = lax.dot_general(a_scr[:, :r0], w16, dn,
                          preferred_element_type=jnp.float32)
    else:
      z = (lax.dot_general(a_scr[:, :r0 - BS], w16[:r0 - BS], dn,
                           preferred_element_type=jnp.float32)
           + lax.dot_general(hprev, w16[r0 - BS:], dn,
                             preferred_element_type=jnp.float32))
    h = 1.0 / (1.0 + jnp.exp(-(z + b_ref[pl.ds(I + l * BS, BS)])))
    hprev = h.astype(jnp.bfloat16)
    a_scr[:, r0:r0 + BS] = hprev
    if l >= L - O // BS:
      lo = (l - (L - O // BS)) * BS
      o_ref[:, lo:lo + BS] = h


def _tc_scan(x, w2, b):
  return pl.pallas_call(
      _tc_scan_body,
      in_specs=[
          pl.BlockSpec((B, I), lambda: (0, 0)),
          pl.BlockSpec(memory_space=pltpu.MemorySpace.HBM),
          pl.BlockSpec((N,), lambda: (0,)),
      ],
      out_specs=pl.BlockSpec((B, O), lambda: (0, 0)),
      out_shape=jax.ShapeDtypeStruct((B, O), jnp.float32),
      scratch_shapes=(
          [pltpu.VMEM((B, N), jnp.bfloat16)]
          + [pltpu.VMEM((R0[l], 128), jnp.float32) for l in range(L)]
          + [pltpu.SemaphoreType.DMA for _ in range(L)]
      ),
  )(x, w2, b)


def kernel(x, W_idx, W_val, b):
  wflat = _sc_scatter(W_idx, W_val)
  w2 = wflat.reshape(OUT_FLOATS // 128, 128)
  return _tc_scan(x, w2, b)


# speedup vs baseline: 1.0400x; 1.0310x over previous
import jax
import jax.numpy as jnp
from jax import lax
from jax.experimental import pallas as pl
from jax.experimental.pallas import tpu as pltpu
from jax.experimental.pallas import tpu_sc as plsc

N = 2048
I = 256
O = 256
B = 512
BS = 128
L = (N - I) // BS
NH = N - I
E = NH * 32

R0 = [I + BS * l for l in range(L)]
OFF128 = [64 * l * l + 192 * l for l in range(L)]
OFF = [o * BS for o in OFF128]
TOT = OFF[-1] + R0[-1] * BS

NC = 2
NS = 16
CUT = 983040
SIZE0 = CUT
SIZE1 = TOT - CUT
PAD = 256
OUT_FLOATS = TOT
EPW = E // NS
ROWS128 = EPW // 128
TPC0 = SIZE0 // NS
TPC1 = SIZE1 // NS
ZB = 5120
ZITER = TPC0 // ZB


def _sc_scatter_body(widx_hbm, val_hbm, out_hbm,
                     dstw_v, srcw_v, val_v, idx_v, zbuf_v, wsp,
                     ldsem, zsem, ssem):
  c = lax.axis_index("c")
  s = lax.axis_index("s")

  ld_d = pltpu.async_copy(widx_hbm.at[0, pl.ds(s * EPW, EPW)], dstw_v, ldsem)
  ld_s = pltpu.async_copy(widx_hbm.at[1, pl.ds(s * EPW, EPW)], srcw_v, ldsem)
  ld_v = pltpu.async_copy(val_hbm.at[pl.ds(s * EPW, EPW)], val_v, ldsem)

  @plsc.parallel_loop(0, ZB, step=16, unroll=4)
  def _zfill(i):
    zbuf_v[pl.ds(i, 16)] = jnp.zeros((16,), jnp.float32)

  zcps = []
  for j in range(ZITER):
    zcps.append(pltpu.async_copy(
        zbuf_v, wsp.at[pl.ds(s * TPC0 + j * ZB, ZB)], zsem))

  ld_d.wait()
  ld_s.wait()
  base = c * CUT
  size_c = jnp.where(c == 0, SIZE0, SIZE1)
  trash = jnp.arange(16, dtype=jnp.int32) + (TPC0 * NS + s * 16)

  @plsc.parallel_loop(0, ROWS128 * 8, step=1, unroll=2)
  def _cvt(i):
    j = i // 8
    k = i - j * 8
    d = dstw_v[pl.ds(i * 16, 16)] - I
    sr = srcw_v[pl.ds(i * 16, 16)]
    lyr = jnp.right_shift(d, 7)
    flat = (8192 * lyr + 24576) * lyr + sr * 128 + (d - 128 * lyr) - base
    ok = (flat >= 0) & (flat < size_c) & (sr < 128 * lyr + I)
    idx_v[j, pl.ds(k * 16, 16)] = jnp.where(ok, flat, trash)

  ld_v.wait()
  for cp in zcps:
    cp.wait()
  plsc.subcore_barrier()

  for j0 in range(0, ROWS128, 14):
    cps = [pltpu.async_copy(val_v.at[pl.ds(j * 128, 128)],
                            wsp.at[idx_v.at[j]], ssem, add=True)
           for j in range(j0, j0 + 14)]
    for cp in cps:
      cp.wait()
  plsc.subcore_barrier()

  @pl.when(c == 0)
  def _out0():
    pltpu.sync_copy(wsp.at[pl.ds(s * TPC0, TPC0)],
                    out_hbm.at[pl.ds(s * TPC0, TPC0)])

  @pl.when(c == 1)
  def _out1():
    pltpu.sync_copy(wsp.at[pl.ds(s * TPC1, TPC1)],
                    out_hbm.at[pl.ds(CUT + s * TPC1, TPC1)])


def _sc_scatter(W_idx, W_val):
  mesh = plsc.VectorSubcoreMesh(core_axis_name="c", subcore_axis_name="s")
  return pl.kernel(
      _sc_scatter_body,
      out_type=jax.ShapeDtypeStruct((OUT_FLOATS,), jnp.float32),
      mesh=mesh,
      scratch_types=[
          pltpu.VMEM((EPW,), jnp.int32),
          pltpu.VMEM((EPW,), jnp.int32),
          pltpu.VMEM((EPW,), jnp.float32),
          pltpu.VMEM((ROWS128, 128), jnp.int32),
          pltpu.VMEM((ZB,), jnp.float32),
          pltpu.VMEM_SHARED((CUT + PAD,), jnp.float32),
          pltpu.SemaphoreType.DMA,
          pltpu.SemaphoreType.DMA,
          pltpu.SemaphoreType.DMA,
      ],
  )(W_idx, W_val)


def _tc_scan_body(x_ref, w_hbm, b_ref, o_ref, a_scr, *rest):
  bufs, sems = rest[:L], rest[L:]
  copies = []
  for l in range(L):
    cp = pltpu.make_async_copy(w_hbm.at[pl.ds(OFF128[l], R0[l])],
                               bufs[l], sems[l])
    cp.start()
    copies.append(cp)
  a_scr[:, :I] = x_ref[...].astype(jnp.bfloat16)
  dn = (((1,), (0,)), ((), ()))
  hprev = None
  for l in range(L):
    r0 = R0[l]
    copies[l].wait()
    w16 = bufs[l][...].astype(jnp.bfloat16)
    if l == 0:
      z = lax.dot_general(a_scr[:, :r0], w16, dn,
                          preferred_element_type=jnp.float32)
    else:
      z = (lax.dot_general(a_scr[:, :r0 - BS], w16[:r0 - BS], dn,
                           preferred_element_type=jnp.float32)
           + lax.dot_general(hprev, w16[r0 - BS:], dn,
                             preferred_element_type=jnp.float32))
    h = 1.0 / (1.0 + jnp.exp(-(z + b_ref[pl.ds(I + l * BS, BS)])))
    hprev = h.astype(jnp.bfloat16)
    a_scr[:, r0:r0 + BS] = hprev
    if l >= L - O // BS:
      lo = (l - (L - O // BS)) * BS
      o_ref[:, lo:lo + BS] = h


def _tc_scan(x, w2, b):
  return pl.pallas_call(
      _tc_scan_body,
      in_specs=[
          pl.BlockSpec((B, I), lambda: (0, 0)),
          pl.BlockSpec(memory_space=pltpu.MemorySpace.HBM),
          pl.BlockSpec((N,), lambda: (0,)),
      ],
      out_specs=pl.BlockSpec((B, O), lambda: (0, 0)),
      out_shape=jax.ShapeDtypeStruct((B, O), jnp.float32),
      scratch_shapes=(
          [pltpu.VMEM((B, N), jnp.bfloat16)]
          + [pltpu.VMEM((R0[l], 128), jnp.float32) for l in range(L)]
          + [pltpu.SemaphoreType.DMA for _ in range(L)]
      ),
  )(x, w2, b)


def kernel(x, W_idx, W_val, b):
  wflat = _sc_scatter(W_idx, W_val)
  w2 = wflat.reshape(OUT_FLOATS // 128, 128)
  return _tc_scan(x, w2, b)


# speedup vs baseline: 1.0410x; 1.0010x over previous
import jax
import jax.numpy as jnp
from jax import lax
from jax.experimental import pallas as pl
from jax.experimental.pallas import tpu as pltpu
from jax.experimental.pallas import tpu_sc as plsc

N = 2048
I = 256
O = 256
B = 512
BS = 128
L = (N - I) // BS
NH = N - I
E = NH * 32

R0 = [I + BS * l for l in range(L)]
OFF128 = [64 * l * l + 192 * l for l in range(L)]
OFF = [o * BS for o in OFF128]
TOT = OFF[-1] + R0[-1] * BS

NC = 2
NS = 16
CUT = 983040
SIZE0 = CUT
SIZE1 = TOT - CUT
PAD = 1024
OUT_FLOATS = TOT
EPW = E // NS
ROWS128 = EPW // 128
TPC0 = SIZE0 // NS
TPC1 = SIZE1 // NS
ZB = 5120
ZITER = TPC0 // ZB


def _sc_scatter_body(widx_hbm, val_hbm, out_hbm,
                     dstw_v, srcw_v, val_v, idx_v, zbuf_v, wsp,
                     ldsem, zsem, ssem):
  c = lax.axis_index("c")
  s = lax.axis_index("s")

  ld_d = pltpu.async_copy(widx_hbm.at[0, pl.ds(s * EPW, EPW)], dstw_v, ldsem)
  ld_s = pltpu.async_copy(widx_hbm.at[1, pl.ds(s * EPW, EPW)], srcw_v, ldsem)
  ld_v = pltpu.async_copy(val_hbm.at[pl.ds(s * EPW, EPW)], val_v, ldsem)

  @plsc.parallel_loop(0, ZB, step=16, unroll=4)
  def _zfill(i):
    zbuf_v[pl.ds(i, 16)] = jnp.zeros((16,), jnp.float32)

  zcps = []
  for j in range(ZITER):
    zcps.append(pltpu.async_copy(
        zbuf_v, wsp.at[pl.ds(s * TPC0 + j * ZB, ZB)], zsem))

  ld_d.wait()
  ld_s.wait()
  base = c * CUT
  size_c = jnp.where(c == 0, SIZE0, SIZE1)
  trash_base = TPC0 * NS + s * 64 + jnp.arange(16, dtype=jnp.int32)

  @plsc.parallel_loop(0, ROWS128 * 8, step=1, unroll=2)
  def _cvt(i):
    j = i // 8
    k = i - j * 8
    d = dstw_v[pl.ds(i * 16, 16)] - I
    sr = srcw_v[pl.ds(i * 16, 16)]
    lyr = jnp.right_shift(d, 7)
    flat = (8192 * lyr + 24576) * lyr + sr * 128 + (d - 128 * lyr) - base
    ok = (flat >= 0) & (flat < size_c) & (sr < 128 * lyr + I)
    trash = trash_base + (i & 3) * 16
    idx_v[j, pl.ds(k * 16, 16)] = jnp.where(ok, flat, trash)

  ld_v.wait()
  for cp in zcps:
    cp.wait()
  plsc.subcore_barrier()

  for j0 in range(0, ROWS128, 14):
    cps = [pltpu.async_copy(val_v.at[pl.ds(j * 128, 128)],
                            wsp.at[idx_v.at[j]], ssem, add=True)
           for j in range(j0, j0 + 14)]
    for cp in cps:
      cp.wait()
  plsc.subcore_barrier()

  @pl.when(c == 0)
  def _out0():
    pltpu.sync_copy(wsp.at[pl.ds(s * TPC0, TPC0)],
                    out_hbm.at[pl.ds(s * TPC0, TPC0)])

  @pl.when(c == 1)
  def _out1():
    pltpu.sync_copy(wsp.at[pl.ds(s * TPC1, TPC1)],
                    out_hbm.at[pl.ds(CUT + s * TPC1, TPC1)])


def _sc_scatter(W_idx, W_val):
  mesh = plsc.VectorSubcoreMesh(core_axis_name="c", subcore_axis_name="s")
  return pl.kernel(
      _sc_scatter_body,
      out_type=jax.ShapeDtypeStruct((OUT_FLOATS,), jnp.float32),
      mesh=mesh,
      scratch_types=[
          pltpu.VMEM((EPW,), jnp.int32),
          pltpu.VMEM((EPW,), jnp.int32),
          pltpu.VMEM((EPW,), jnp.float32),
          pltpu.VMEM((ROWS128, 128), jnp.int32),
          pltpu.VMEM((ZB,), jnp.float32),
          pltpu.VMEM_SHARED((CUT + PAD,), jnp.float32),
          pltpu.SemaphoreType.DMA,
          pltpu.SemaphoreType.DMA,
          pltpu.SemaphoreType.DMA,
      ],
  )(W_idx, W_val)


def _tc_scan_body(x_ref, w_hbm, b_ref, o_ref, a_scr, *rest):
  bufs, sems = rest[:L], rest[L:]
  copies = []
  for l in range(L):
    cp = pltpu.make_async_copy(w_hbm.at[pl.ds(OFF128[l], R0[l])],
                               bufs[l], sems[l])
    cp.start()
    copies.append(cp)
  a_scr[:, :I] = x_ref[...].astype(jnp.bfloat16)
  dn = (((1,), (0,)), ((), ()))
  hprev = None
  for l in range(L):
    r0 = R0[l]
    copies[l].wait()
    w16 = bufs[l][...].astype(jnp.bfloat16)
    if l == 0:
      z = lax.dot_general(a_scr[:, :r0], w16, dn,
                          preferred_element_type=jnp.float32)
    else:
      z = (lax.dot_general(a_scr[:, :r0 - BS], w16[:r0 - BS], dn,
                           preferred_element_type=jnp.float32)
           + lax.dot_general(hprev, w16[r0 - BS:], dn,
                             preferred_element_type=jnp.float32))
    h = 1.0 / (1.0 + jnp.exp(-(z + b_ref[pl.ds(I + l * BS, BS)])))
    hprev = h.astype(jnp.bfloat16)
    a_scr[:, r0:r0 + BS] = hprev
    if l >= L - O // BS:
      lo = (l - (L - O // BS)) * BS
      o_ref[:, lo:lo + BS] = h


def _tc_scan(x, w2, b):
  return pl.pallas_call(
      _tc_scan_body,
      in_specs=[
          pl.BlockSpec((B, I), lambda: (0, 0)),
          pl.BlockSpec(memory_space=pltpu.MemorySpace.HBM),
          pl.BlockSpec((N,), lambda: (0,)),
      ],
      out_specs=pl.BlockSpec((B, O), lambda: (0, 0)),
      out_shape=jax.ShapeDtypeStruct((B, O), jnp.float32),
      scratch_shapes=(
          [pltpu.VMEM((B, N), jnp.bfloat16)]
          + [pltpu.VMEM((R0[l], 128), jnp.float32) for l in range(L)]
          + [pltpu.SemaphoreType.DMA for _ in range(L)]
      ),
  )(x, w2, b)


def kernel(x, W_idx, W_val, b):
  wflat = _sc_scatter(W_idx, W_val)
  w2 = wflat.reshape(OUT_FLOATS // 128, 128)
  return _tc_scan(x, w2, b)
